# baseline (speedup 1.0000x reference)
import dataclasses

import jax
import jax.numpy as jnp
import numpy as np
from jax.experimental import pallas as pl
from jax.experimental.pallas import tpu as pltpu
from jax.experimental.pallas import tpu_sc as plsc

D = 2048
ROWS = 4 * 4096
RB = 8
LANES = 16
NBANKS = 16

_COMPILER_PARAMS = pltpu.CompilerParams()
if "needs_layout_passes" in pltpu.CompilerParams.__dataclass_fields__:
    _COMPILER_PARAMS = dataclasses.replace(
        _COMPILER_PARAMS, needs_layout_passes=False
    )


def _perfect_matching(cnt):
    match_of_b = [-1] * NBANKS

    def try_assign(a, seen):
        for b in range(NBANKS):
            if cnt[a][b] > 0 and not seen[b]:
                seen[b] = True
                if match_of_b[b] < 0 or try_assign(match_of_b[b], seen):
                    match_of_b[b] = a
                    return True
        return False

    for a in range(NBANKS):
        assert try_assign(a, [False] * NBANKS)
    b_of_a = [-1] * NBANKS
    for b, a in enumerate(match_of_b):
        b_of_a[a] = b
    return b_of_a


def _conflict_free_order(perm):
    buckets = [[[] for _ in range(NBANKS)] for _ in range(NBANKS)]
    for j in range(D):
        buckets[j % NBANKS][int(perm[j]) % NBANKS].append(j)
    cnt = [[len(buckets[a][b]) for b in range(NBANKS)] for a in range(NBANKS)]
    order = []
    for _ in range(D // NBANKS):
        b_of_a = _perfect_matching(cnt)
        for a in range(NBANKS):
            b = b_of_a[a]
            order.append(buckets[a][b].pop())
            cnt[a][b] -= 1
    return np.asarray(order, dtype=np.int32)


_FIXED_PERM = np.random.RandomState(0).permutation(D)
_DST_ORDER = jnp.asarray(_conflict_free_order(_FIXED_PERM))


NTC = 8192
BT = 512


def _tc_matmul_permute(x_tc, p_onehot):
    def body(x_ref, p_ref, o_ref):
        o_ref[...] = jnp.dot(
            x_ref[...].astype(jnp.bfloat16),
            p_ref[...],
            preferred_element_type=jnp.float32,
        )

    return pl.pallas_call(
        body,
        grid=(NTC // BT,),
        in_specs=[
            pl.BlockSpec((BT, D), lambda i: (i, 0)),
            pl.BlockSpec((D, D), lambda i: (0, 0)),
        ],
        out_specs=pl.BlockSpec((BT, D), lambda i: (i, 0)),
        out_shape=jax.ShapeDtypeStruct((NTC, D), jnp.float32),
    )(x_tc, p_onehot)


@jax.jit
def _permute_rows(flat, src_idx, dst_idx):
    mesh = plsc.VectorSubcoreMesh(core_axis_name="core",
                                  subcore_axis_name="subcore")

    @pl.kernel(
        out_type=jax.ShapeDtypeStruct((ROWS - NTC, D), jnp.float32),
        mesh=mesh,
        compiler_params=_COMPILER_PARAMS,
        scratch_types=[
            pltpu.VMEM((D,), jnp.int32),
            pltpu.VMEM((D,), jnp.int32),
            pltpu.SemaphoreType.DMA,
        ],
    )
    def kern(x_hbm, s_hbm, d_hbm, o_hbm, src_vmem, dst_vmem, sem):
        cp1 = pltpu.async_copy(s_hbm, src_vmem, sem)
        cp2 = pltpu.async_copy(d_hbm, dst_vmem, sem)
        cp1.wait()
        cp2.wait()

        def body(in_vmem, out_vmem):
            @plsc.parallel_loop(0, D // LANES)
            def _grp(k):
                base = k * LANES
                src = src_vmem[pl.ds(base, LANES)]
                dst = dst_vmem[pl.ds(base, LANES)]
                for r in range(RB):
                    r_vec = jnp.full((LANES,), r, jnp.int32)
                    v = plsc.load_gather(in_vmem, [r_vec, src])
                    plsc.store_scatter(out_vmem, [r_vec, dst], v)

        pltpu.emit_pipeline(
            body,
            grid=((ROWS - NTC) // RB,),
            in_specs=[pl.BlockSpec((RB, D), index_map=lambda i: (i, 0))],
            out_specs=[pl.BlockSpec((RB, D), index_map=lambda i: (i, 0))],
            core_axis_name=("core", "subcore"),
            dimension_semantics=(pltpu.PARALLEL,),
        )(x_hbm, o_hbm)

    out_sc = kern(flat[NTC:], src_idx, dst_idx)
    p_onehot = (
        jax.lax.broadcasted_iota(jnp.int32, (D, D), 0)
        == jnp.take(src_idx, jnp.argsort(dst_idx))[None, :]
    ).astype(jnp.bfloat16)
    out_tc = _tc_matmul_permute(flat[:NTC], p_onehot)
    return jnp.concatenate([out_tc, out_sc], axis=0)


def kernel(target, permutation):
    b, s, d = target.shape
    dst_idx = _DST_ORDER
    src_idx = permutation[dst_idx]
    out = _permute_rows(target.reshape(b * s, d), src_idx, dst_idx)
    return out.reshape(b, s, d)


# speedup vs baseline: 2.7000x; 2.7000x over previous
import dataclasses

import jax
import jax.numpy as jnp
import numpy as np
from jax.experimental import pallas as pl
from jax.experimental.pallas import tpu as pltpu
from jax.experimental.pallas import tpu_sc as plsc

D = 2048
ROWS = 4 * 4096
RB = 8
LANES = 16
NBANKS = 16

_COMPILER_PARAMS = pltpu.CompilerParams()
if "needs_layout_passes" in pltpu.CompilerParams.__dataclass_fields__:
    _COMPILER_PARAMS = dataclasses.replace(
        _COMPILER_PARAMS, needs_layout_passes=False
    )


def _perfect_matching(cnt):
    match_of_b = [-1] * NBANKS

    def try_assign(a, seen):
        for b in range(NBANKS):
            if cnt[a][b] > 0 and not seen[b]:
                seen[b] = True
                if match_of_b[b] < 0 or try_assign(match_of_b[b], seen):
                    match_of_b[b] = a
                    return True
        return False

    for a in range(NBANKS):
        assert try_assign(a, [False] * NBANKS)
    b_of_a = [-1] * NBANKS
    for b, a in enumerate(match_of_b):
        b_of_a[a] = b
    return b_of_a


def _conflict_free_order(perm):
    buckets = [[[] for _ in range(NBANKS)] for _ in range(NBANKS)]
    for j in range(D):
        buckets[j % NBANKS][int(perm[j]) % NBANKS].append(j)
    cnt = [[len(buckets[a][b]) for b in range(NBANKS)] for a in range(NBANKS)]
    order = []
    for _ in range(D // NBANKS):
        b_of_a = _perfect_matching(cnt)
        for a in range(NBANKS):
            b = b_of_a[a]
            order.append(buckets[a][b].pop())
            cnt[a][b] -= 1
    return np.asarray(order, dtype=np.int32)


_FIXED_PERM = np.random.RandomState(0).permutation(D)
_DST_ORDER = jnp.asarray(_conflict_free_order(_FIXED_PERM))


@jax.jit
def _permute_rows(flat, perm, dst_idx):
    mesh = plsc.VectorSubcoreMesh(core_axis_name="core",
                                  subcore_axis_name="subcore")

    @pl.kernel(
        out_type=jax.ShapeDtypeStruct((ROWS, D), jnp.float32),
        mesh=mesh,
        compiler_params=_COMPILER_PARAMS,
        scratch_types=[
            pltpu.VMEM((D,), jnp.int32),
            pltpu.VMEM((D,), jnp.int32),
            pltpu.SemaphoreType.DMA,
        ],
    )
    def kern(x_hbm, p_hbm, d_hbm, o_hbm, perm_vmem, dst_vmem, sem):
        cp1 = pltpu.async_copy(p_hbm, perm_vmem, sem)
        cp2 = pltpu.async_copy(d_hbm, dst_vmem, sem)
        cp1.wait()
        cp2.wait()

        def body(in_vmem, out_vmem):
            @plsc.parallel_loop(0, D // LANES)
            def _grp(k):
                base = k * LANES
                dst = dst_vmem[pl.ds(base, LANES)]
                src = plsc.load_gather(perm_vmem, [dst])
                for r in range(RB):
                    r_vec = jnp.full((LANES,), r, jnp.int32)
                    v = plsc.load_gather(in_vmem, [r_vec, src])
                    plsc.store_scatter(out_vmem, [r_vec, dst], v)

        pltpu.emit_pipeline(
            body,
            grid=(ROWS // RB,),
            in_specs=[pl.BlockSpec((RB, D), index_map=lambda i: (i, 0))],
            out_specs=[pl.BlockSpec((RB, D), index_map=lambda i: (i, 0))],
            core_axis_name=("core", "subcore"),
            dimension_semantics=(pltpu.PARALLEL,),
        )(x_hbm, o_hbm)

    return kern(flat, perm, dst_idx)


def kernel(target, permutation):
    b, s, d = target.shape
    out = _permute_rows(target.reshape(b * s, d), permutation, _DST_ORDER)
    return out.reshape(b, s, d)


# speedup vs baseline: 2.7648x; 1.0240x over previous
import dataclasses

import jax
import jax.numpy as jnp
import numpy as np
from jax.experimental import pallas as pl
from jax.experimental.pallas import tpu as pltpu
from jax.experimental.pallas import tpu_sc as plsc

D = 2048
ROWS = 4 * 4096
RB = 8
LANES = 16
NBANKS = 16

_COMPILER_PARAMS = pltpu.CompilerParams()
if "needs_layout_passes" in pltpu.CompilerParams.__dataclass_fields__:
    _COMPILER_PARAMS = dataclasses.replace(
        _COMPILER_PARAMS, needs_layout_passes=False
    )


def _perfect_matching(cnt):
    match_of_b = [-1] * NBANKS

    def try_assign(a, seen):
        for b in range(NBANKS):
            if cnt[a][b] > 0 and not seen[b]:
                seen[b] = True
                if match_of_b[b] < 0 or try_assign(match_of_b[b], seen):
                    match_of_b[b] = a
                    return True
        return False

    for a in range(NBANKS):
        assert try_assign(a, [False] * NBANKS)
    b_of_a = [-1] * NBANKS
    for b, a in enumerate(match_of_b):
        b_of_a[a] = b
    return b_of_a


def _conflict_free_order(perm):
    buckets = [[[] for _ in range(NBANKS)] for _ in range(NBANKS)]
    for j in range(D):
        buckets[j % NBANKS][int(perm[j]) % NBANKS].append(j)
    cnt = [[len(buckets[a][b]) for b in range(NBANKS)] for a in range(NBANKS)]
    order = []
    for _ in range(D // NBANKS):
        b_of_a = _perfect_matching(cnt)
        for a in range(NBANKS):
            b = b_of_a[a]
            order.append(buckets[a][b].pop())
            cnt[a][b] -= 1
    return np.asarray(order, dtype=np.int32)


_FIXED_PERM = np.random.RandomState(0).permutation(D)
_DST_ORDER = jnp.asarray(_conflict_free_order(_FIXED_PERM))


@jax.jit
def _permute_rows(flat, perm, dst_idx):
    mesh = plsc.VectorSubcoreMesh(core_axis_name="core",
                                  subcore_axis_name="subcore")

    @pl.kernel(
        out_type=jax.ShapeDtypeStruct((ROWS, D), jnp.float32),
        mesh=mesh,
        compiler_params=_COMPILER_PARAMS,
        scratch_types=[
            pltpu.VMEM((D,), jnp.int32),
            pltpu.VMEM((D,), jnp.int32),
            pltpu.SemaphoreType.DMA,
        ],
    )
    def kern(x_hbm, p_hbm, d_hbm, o_hbm, perm_vmem, dst_vmem, sem):
        cp1 = pltpu.async_copy(p_hbm, perm_vmem, sem)
        cp2 = pltpu.async_copy(d_hbm, dst_vmem, sem)
        cp1.wait()
        cp2.wait()

        def body(in_vmem, out_vmem):
            @plsc.parallel_loop(0, D // LANES)
            def _grp(k):
                base = k * LANES
                dst = dst_vmem[pl.ds(base, LANES)]
                src = plsc.load_gather(perm_vmem, [dst])
                for r in range(RB):
                    r_vec = jnp.full((LANES,), r, jnp.int32)
                    v = plsc.load_gather(in_vmem, [r_vec, src])
                    plsc.store_scatter(out_vmem, [r_vec, dst], v)

        pltpu.emit_pipeline(
            body,
            grid=(ROWS // RB,),
            in_specs=[pl.BlockSpec((RB, D), index_map=lambda i: (i, 0),
                                   pipeline_mode=pl.Buffered(buffer_count=3))],
            out_specs=[pl.BlockSpec((RB, D), index_map=lambda i: (i, 0))],
            core_axis_name=("core", "subcore"),
            dimension_semantics=(pltpu.PARALLEL,),
        )(x_hbm, o_hbm)

    return kern(flat, perm, dst_idx)


def kernel(target, permutation):
    b, s, d = target.shape
    out = _permute_rows(target.reshape(b * s, d), permutation, _DST_ORDER)
    return out.reshape(b, s, d)


# speedup vs baseline: 2.7806x; 1.0057x over previous
import dataclasses

import jax
import jax.numpy as jnp
import numpy as np
from jax.experimental import pallas as pl
from jax.experimental.pallas import tpu as pltpu
from jax.experimental.pallas import tpu_sc as plsc

D = 2048
ROWS = 4 * 4096
RB = 8
LANES = 16
NBANKS = 16

_COMPILER_PARAMS = pltpu.CompilerParams()
if "needs_layout_passes" in pltpu.CompilerParams.__dataclass_fields__:
    _COMPILER_PARAMS = dataclasses.replace(
        _COMPILER_PARAMS, needs_layout_passes=False
    )


def _perfect_matching(cnt):
    match_of_b = [-1] * NBANKS

    def try_assign(a, seen):
        for b in range(NBANKS):
            if cnt[a][b] > 0 and not seen[b]:
                seen[b] = True
                if match_of_b[b] < 0 or try_assign(match_of_b[b], seen):
                    match_of_b[b] = a
                    return True
        return False

    for a in range(NBANKS):
        assert try_assign(a, [False] * NBANKS)
    b_of_a = [-1] * NBANKS
    for b, a in enumerate(match_of_b):
        b_of_a[a] = b
    return b_of_a


def _conflict_free_order(perm):
    buckets = [[[] for _ in range(NBANKS)] for _ in range(NBANKS)]
    for j in range(D):
        buckets[j % NBANKS][int(perm[j]) % NBANKS].append(j)
    cnt = [[len(buckets[a][b]) for b in range(NBANKS)] for a in range(NBANKS)]
    order = []
    for _ in range(D // NBANKS):
        b_of_a = _perfect_matching(cnt)
        for a in range(NBANKS):
            b = b_of_a[a]
            order.append(buckets[a][b].pop())
            cnt[a][b] -= 1
    return np.asarray(order, dtype=np.int32)


_FIXED_PERM = np.random.RandomState(0).permutation(D)
_DST_ORDER = jnp.asarray(_conflict_free_order(_FIXED_PERM))


@jax.jit
def _permute_rows(flat, perm, dst_idx):
    mesh = plsc.VectorSubcoreMesh(core_axis_name="core",
                                  subcore_axis_name="subcore")

    @pl.kernel(
        out_type=jax.ShapeDtypeStruct((ROWS, D), jnp.float32),
        mesh=mesh,
        compiler_params=_COMPILER_PARAMS,
        scratch_types=[
            pltpu.VMEM((D,), jnp.int32),
            pltpu.VMEM((D,), jnp.int32),
            pltpu.SemaphoreType.DMA,
        ],
    )
    def kern(x_hbm, p_hbm, d_hbm, o_hbm, perm_vmem, dst_vmem, sem):
        cp1 = pltpu.async_copy(p_hbm, perm_vmem, sem)
        cp2 = pltpu.async_copy(d_hbm, dst_vmem, sem)
        cp1.wait()
        cp2.wait()

        def body(in_vmem, out_vmem):
            @plsc.parallel_loop(0, D // LANES)
            def _grp(k):
                base = k * LANES
                dst = dst_vmem[pl.ds(base, LANES)]
                src = plsc.load_gather(perm_vmem, [dst])
                for r in range(RB):
                    r_vec = jnp.full((LANES,), r, jnp.int32)
                    v = plsc.load_gather(in_vmem, [r_vec, src])
                    plsc.store_scatter(out_vmem, [r_vec, dst], v)

        pltpu.emit_pipeline(
            body,
            grid=(ROWS // RB,),
            in_specs=[pl.BlockSpec((RB, D), index_map=lambda i: (i, 0),
                                   pipeline_mode=pl.Buffered(buffer_count=4))],
            out_specs=[pl.BlockSpec((RB, D), index_map=lambda i: (i, 0))],
            core_axis_name=("core", "subcore"),
            dimension_semantics=(pltpu.PARALLEL,),
        )(x_hbm, o_hbm)

    return kern(flat, perm, dst_idx)


def kernel(target, permutation):
    b, s, d = target.shape
    out = _permute_rows(target.reshape(b * s, d), permutation, _DST_ORDER)
    return out.reshape(b, s, d)
